# baseline (speedup 1.0000x reference)
import functools

import jax
import jax.numpy as jnp
from jax import lax
from jax.experimental import pallas as pl
from jax.experimental.pallas import tpu as pltpu
from jax.experimental.pallas import tpu_sc as plsc

VOCAB = 1000000
EMBED_DIM = 64
BATCH = 4096
SEQ_LEN = 200

NUM_CORES = 2
NUM_SUBCORES = 16
NW = NUM_CORES * NUM_SUBCORES

TCH = 400
NCHUNKS = VOCAB // TCH

BT = BATCH // NW
NUNIT = SEQ_LEN
GN = 4


def _sc_mesh():
    return plsc.VectorSubcoreMesh(
        core_axis_name="c", subcore_axis_name="s",
        num_cores=NUM_CORES, num_subcores=NUM_SUBCORES)


@jax.jit
def _embed(text_t, tab_t):
    @functools.partial(
        pl.kernel,
        mesh=_sc_mesh(),
        out_type=jax.ShapeDtypeStruct((VOCAB, EMBED_DIM), jnp.float32),
        scratch_types=[
            pltpu.VMEM((2, EMBED_DIM, TCH), jnp.float32),
            pltpu.VMEM((TCH, EMBED_DIM), jnp.float32),
            pltpu.SemaphoreType.DMA((2,)),
            pltpu.SemaphoreType.DMA,
        ],
        compiler_params=pltpu.CompilerParams(use_tc_tiling_on_sc=False, needs_layout_passes=False),
    )
    def k1(tab_hbm, out_hbm, tin, tout, rsem, wsem):
        wid = lax.axis_index("s") * NUM_CORES + lax.axis_index("c")
        nw_chunks = (NCHUNKS - wid + NW - 1) // NW
        iota = lax.iota(jnp.int32, 16)

        def lo_of(t):
            return (wid + NW * t) * TCH

        def read_start(t, h):
            pltpu.async_copy(tab_hbm.at[:, pl.ds(lo_of(t), TCH)],
                             tin.at[h], rsem.at[h])

        def read_wait(h):
            pltpu.make_async_copy(tab_hbm.at[:, pl.ds(0, TCH)],
                                  tin.at[h], rsem.at[h]).wait()

        def write_start(t):
            pltpu.async_copy(tout, out_hbm.at[pl.ds(lo_of(t), TCH)], wsem)

        def write_wait():
            pltpu.make_async_copy(tout, out_hbm.at[pl.ds(0, TCH)],
                                  wsem).wait()

        def transpose(h):
            def dbody(d, carry):
                dvec = jnp.full((16,), d, jnp.int32)
                for j in range(TCH // 16):
                    v = tin[h, d, pl.ds(16 * j, 16)]
                    plsc.store_scatter(tout, [iota + 16 * j, dvec], v)
                return carry
            lax.fori_loop(0, EMBED_DIM, dbody, 0)

        def do_chunk(t, h, first, last_guard):
            read_wait(h)
            if not first:
                write_wait()
            transpose(h)
            write_start(t)

            @pl.when(t + 2 < nw_chunks)
            def _():
                read_start(t + 2, h)

        read_start(0, 0)

        @pl.when(nw_chunks > 1)
        def _():
            read_start(1, 1)

        def body(p, carry):
            t0 = 2 * p

            @pl.when(t0 < nw_chunks)
            def _():
                do_chunk(t0, 0, False, True)

            @pl.when(t0 + 1 < nw_chunks)
            def _():
                do_chunk(t0 + 1, 1, False, True)

            return carry

        do_chunk(0, 0, True, True)

        @pl.when(nw_chunks > 1)
        def _():
            do_chunk(1, 1, False, True)

        lax.fori_loop(1, (NCHUNKS // NW + 2) // 2, body, 0)
        write_wait()

    table_rm = k1(tab_t)

    @functools.partial(
        pl.kernel,
        mesh=_sc_mesh(),
        out_type=jax.ShapeDtypeStruct(
            (SEQ_LEN, EMBED_DIM // 8, NW, 8, BT), jnp.float32),
        scratch_types=[
            pltpu.VMEM((NUNIT, BT), jnp.int32),
            pltpu.VMEM((GN, BT, EMBED_DIM), jnp.float32),
            pltpu.VMEM((GN, EMBED_DIM // 8, 8, BT), jnp.float32),
            pltpu.SemaphoreType.DMA((GN,)),
            pltpu.SemaphoreType.DMA((GN,)),
        ],
        compiler_params=pltpu.CompilerParams(use_tc_tiling_on_sc=False, needs_layout_passes=False),
    )
    def k2(text_hbm, tab_hbm, out_hbm, idx_v, grows, otile, gsem, wsem):
        wid = lax.axis_index("s") * NUM_CORES + lax.axis_index("c")
        iota = lax.iota(jnp.int32, 16)
        dts = [lax.shift_right_logical(iota + 16 * k, 3) for k in range(4)]
        drs = [lax.bitwise_and(iota + 16 * k, 7) for k in range(4)]

        pltpu.sync_copy(text_hbm.at[:, pl.ds(wid * BT, BT)], idx_v)

        def gather_start(u, h):
            pltpu.async_copy(tab_hbm.at[idx_v.at[u]], grows.at[h],
                             gsem.at[h])

        def gather_wait(h):
            pltpu.make_async_copy(tab_hbm.at[idx_v.at[0]], grows.at[h],
                                  gsem.at[h]).wait()

        def write_start(u, h):
            for dt in range(EMBED_DIM // 8):
                pltpu.async_copy(otile.at[h, dt], out_hbm.at[u, dt, wid],
                                 wsem.at[h])

        def write_wait(h):
            for dt in range(EMBED_DIM // 8):
                pltpu.make_async_copy(otile.at[h, dt],
                                      out_hbm.at[0, dt, wid],
                                      wsem.at[h]).wait()

        def transform(h):
            def bbody(i, carry):
                for q in range(8):
                    bc = 8 * i + q
                    bcv = jnp.full((16,), bc, jnp.int32)
                    for k in range(4):
                        v = grows[h, bc, pl.ds(16 * k, 16)]
                        plsc.store_scatter(otile.at[h], [dts[k], drs[k], bcv], v)
                return carry
            lax.fori_loop(0, BT // 8, bbody, 0)

        for h in range(GN):
            gather_start(h, h)

        def body(p, carry):
            for h in range(GN):
                u = GN * p + h
                gather_wait(h)

                @pl.when(p >= 1)
                def _():
                    write_wait(h)

                transform(h)
                write_start(u, h)

                @pl.when(p < NUNIT // GN - 1)
                def _():
                    gather_start(u + GN, h)

            return carry

        lax.fori_loop(0, NUNIT // GN, body, 0)
        for h in range(GN):
            write_wait(h)

    out5 = k2(text_t, table_rm)
    return out5


def kernel(text, table):
    out5 = _embed(text.T, table.T)
    return out5.transpose(2, 4, 0, 1, 3).reshape(BATCH, SEQ_LEN, EMBED_DIM)


# speedup vs baseline: 2.0007x; 2.0007x over previous
import functools

import jax
import jax.numpy as jnp
from jax import lax
from jax.experimental import pallas as pl
from jax.experimental.pallas import tpu as pltpu
from jax.experimental.pallas import tpu_sc as plsc

VOCAB = 1000000
EMBED_DIM = 64
BATCH = 4096
SEQ_LEN = 200

NUM_CORES = 2
NUM_SUBCORES = 16
NW = NUM_CORES * NUM_SUBCORES

TCH = 512
NFULL = 1953
TAILW = VOCAB - NFULL * TCH

BT = BATCH // NW
NUNIT = SEQ_LEN
GN = 3


def _sc_mesh():
    return plsc.VectorSubcoreMesh(
        core_axis_name="c", subcore_axis_name="s",
        num_cores=NUM_CORES, num_subcores=NUM_SUBCORES)


@jax.jit
def _embed(ptext_t, off_t, tab_t):
    @functools.partial(
        pl.kernel,
        mesh=_sc_mesh(),
        out_type=jax.ShapeDtypeStruct((VOCAB // 2, 128), jnp.float32),
        scratch_types=[
            pltpu.VMEM((2, EMBED_DIM, TCH), jnp.float32),
            pltpu.VMEM((TCH // 2, 128), jnp.float32),
            pltpu.VMEM((EMBED_DIM, TAILW), jnp.float32),
            pltpu.SemaphoreType.DMA((2,)),
            pltpu.SemaphoreType.DMA,
        ],
        compiler_params=pltpu.CompilerParams(
            use_tc_tiling_on_sc=True, needs_layout_passes=False),
    )
    def k1(tab_hbm, out_hbm, tin, tout, ttail, rsem, wsem):
        wid = lax.axis_index("s") * NUM_CORES + lax.axis_index("c")
        nw_chunks = (NFULL - wid + NW - 1) // NW
        iota = lax.iota(jnp.int32, 16)
        NJ = TCH // 16
        pvecs = [lax.shift_right_logical(iota + 16 * j, 1) for j in range(NJ)]
        ovecs = [lax.shift_left(lax.bitwise_and(iota + 16 * j, 1), 6)
                 for j in range(NJ)]

        def lo_of(t):
            return pl.multiple_of((wid + NW * t) * TCH, TCH)

        def read_start(t, h):
            pltpu.async_copy(tab_hbm.at[:, pl.ds(lo_of(t), TCH)],
                             tin.at[h], rsem.at[h])

        def read_wait(h):
            pltpu.make_async_copy(tab_hbm.at[:, pl.ds(0, TCH)],
                                  tin.at[h], rsem.at[h]).wait()

        def write_start(t):
            pltpu.async_copy(tout,
                             out_hbm.at[pl.ds(pl.multiple_of(lo_of(t) // 2, TCH // 2), TCH // 2)],
                             wsem)

        def write_wait():
            pltpu.make_async_copy(tout,
                                  out_hbm.at[pl.ds(0, TCH // 2)],
                                  wsem).wait()

        def transpose(h):
            def dbody(d, carry):
                vals = [tin[h, d, pl.ds(16 * j, 16)] for j in range(NJ)]
                for j in range(NJ):
                    plsc.store_scatter(tout, [pvecs[j], ovecs[j] + d],
                                       vals[j])
                return carry
            lax.fori_loop(0, EMBED_DIM, dbody, 0)

        def do_chunk(t, h, first):
            read_wait(h)
            if not first:
                write_wait()
            transpose(h)
            write_start(t)

            @pl.when(t + 2 < nw_chunks)
            def _():
                read_start(t + 2, h)

        read_start(0, 0)
        read_start(1, 1)

        def body(p, carry):
            t0 = 2 * p

            @pl.when(t0 < nw_chunks)
            def _():
                do_chunk(t0, 0, False)

            @pl.when(t0 + 1 < nw_chunks)
            def _():
                do_chunk(t0 + 1, 1, False)

            return carry

        do_chunk(0, 0, True)
        do_chunk(1, 1, False)
        lax.fori_loop(1, (NFULL // NW + 2) // 2, body, 0)
        write_wait()

        @pl.when(wid == 0)
        def _():
            pltpu.sync_copy(tab_hbm.at[:, pl.ds(NFULL * TCH, TAILW)], ttail)
            def dbody(d, carry):
                for j in range(TAILW // 16):
                    v = ttail[d, pl.ds(16 * j, 16)]
                    plsc.store_scatter(tout, [pvecs[j], ovecs[j] + d], v)
                return carry
            lax.fori_loop(0, EMBED_DIM, dbody, 0)
            pltpu.sync_copy(tout.at[pl.ds(0, TAILW // 2)],
                            out_hbm.at[pl.ds(NFULL * TCH // 2, TAILW // 2)])

    packed = k1(tab_t).reshape(VOCAB // 2, 1, 128)

    @functools.partial(
        pl.kernel,
        mesh=_sc_mesh(),
        out_type=jax.ShapeDtypeStruct((BATCH, SEQ_LEN, EMBED_DIM),
                                      jnp.float32),
        scratch_types=[
            pltpu.VMEM((NUNIT, BT), jnp.int32),
            pltpu.VMEM((NUNIT, BT), jnp.int32),
            pltpu.VMEM((GN, BT, 1, 128), jnp.float32),
            pltpu.VMEM((GN, BT, 1, EMBED_DIM), jnp.float32),
            pltpu.SemaphoreType.DMA((GN,)),
            pltpu.SemaphoreType.DMA((GN,)),
        ],
        compiler_params=pltpu.CompilerParams(
            use_tc_tiling_on_sc=False, needs_layout_passes=False),
    )
    def k2(pt_hbm, off_hbm, tab_hbm, out_hbm, pidx_v, off_v, grows, obuf,
           gsem, wsem):
        wid = lax.axis_index("s") * NUM_CORES + lax.axis_index("c")
        b0 = wid * BT
        iota = lax.iota(jnp.int32, 16)
        zero16 = jnp.zeros((16,), jnp.int32)
        NG = BT // 16
        bcvecs = [iota + 16 * g for g in range(NG)]

        pltpu.sync_copy(pt_hbm.at[:, pl.ds(b0, BT)], pidx_v)
        pltpu.sync_copy(off_hbm.at[:, pl.ds(b0, BT)], off_v)

        def gather_start(u, h):
            pltpu.async_copy(tab_hbm.at[pidx_v.at[u]], grows.at[h],
                             gsem.at[h])

        def gather_wait(h):
            pltpu.make_async_copy(tab_hbm.at[pidx_v.at[0]], grows.at[h],
                                  gsem.at[h]).wait()

        def write_start(u, h):
            pltpu.async_copy(obuf.at[h],
                             out_hbm.at[pl.ds(b0, BT), pl.ds(u, 1)],
                             wsem.at[h])

        def write_wait(h):
            pltpu.make_async_copy(obuf.at[h],
                                  out_hbm.at[pl.ds(b0, BT), pl.ds(0, 1)],
                                  wsem.at[h]).wait()

        def extract(u, h):
            offs = [off_v[u, pl.ds(16 * g, 16)] for g in range(NG)]

            def dbody(d, carry):
                dvec = jnp.full((16,), d, jnp.int32)
                vals = [plsc.load_gather(grows.at[h], [bcvecs[g], zero16,
                                                       offs[g] + d])
                        for g in range(NG)]
                for g in range(NG):
                    plsc.store_scatter(obuf.at[h],
                                       [bcvecs[g], zero16, dvec], vals[g])
                return carry
            lax.fori_loop(0, EMBED_DIM, dbody, 0)

        for h in range(GN):
            gather_start(h, h)

        def body(p, carry):
            for h in range(GN):
                u = GN * p + h
                gather_wait(h)

                @pl.when(p >= 1)
                def _():
                    write_wait(h)

                extract(u, h)
                write_start(u, h)

                @pl.when(u + GN < NUNIT)
                def _():
                    gather_start(u + GN, h)

            return carry

        nfull = NUNIT // GN
        lax.fori_loop(0, nfull, body, 0)
        for r in range(NUNIT - nfull * GN):
            u = nfull * GN + r
            gather_wait(r)
            write_wait(r)
            extract(u, r)
            write_start(u, r)
        for h in range(GN):
            write_wait(h)

    return k2(ptext_t, off_t, packed)


def kernel(text, table):
    tt = text.T
    ptext_t = jax.lax.shift_right_logical(tt, 1)
    off_t = jax.lax.shift_left(jax.lax.bitwise_and(tt, 1), 6)
    return _embed(ptext_t, off_t, table.T)


# speedup vs baseline: 3.7962x; 1.8975x over previous
import functools

import jax
import jax.numpy as jnp
from jax import lax
from jax.experimental import pallas as pl
from jax.experimental.pallas import tpu as pltpu
from jax.experimental.pallas import tpu_sc as plsc

VOCAB = 1000000
EMBED_DIM = 64
BATCH = 4096
SEQ_LEN = 200

NUM_CORES = 2
NUM_SUBCORES = 16
NW = NUM_CORES * NUM_SUBCORES

TCH = 512
NFULL = VOCAB // TCH
TAILW = VOCAB - NFULL * TCH
BT = BATCH // NW
NUNIT = SEQ_LEN
GN = 4


def _sc_mesh():
    return plsc.VectorSubcoreMesh(
        core_axis_name="c", subcore_axis_name="s",
        num_cores=NUM_CORES, num_subcores=NUM_SUBCORES)


@jax.jit
def _embed(text_t, tab_t, tail_t):
    @functools.partial(
        pl.kernel,
        mesh=_sc_mesh(),
        out_type=jax.ShapeDtypeStruct((VOCAB // 2, 128), jnp.float32),
        scratch_types=[
            pltpu.VMEM((2, EMBED_DIM, TCH), jnp.float32),
            pltpu.VMEM((TCH // 2, 128), jnp.float32),
            pltpu.VMEM((EMBED_DIM, 128), jnp.float32),
            pltpu.SemaphoreType.DMA((2,)),
            pltpu.SemaphoreType.DMA,
        ],
        compiler_params=pltpu.CompilerParams(
            use_tc_tiling_on_sc=True, needs_layout_passes=False),
    )
    def k1(tab_hbm, tail_hbm, out_hbm, tin, tout, ttail, rsem, wsem):
        wid = lax.axis_index("s") * NUM_CORES + lax.axis_index("c")
        nw_chunks = (NFULL - wid + NW - 1) // NW
        iota = lax.iota(jnp.int32, 16)
        NJ = TCH // 16
        pvecs = [lax.shift_right_logical(iota + 16 * j, 1) for j in range(NJ)]
        ovecs = [lax.shift_left(lax.bitwise_and(iota + 16 * j, 1), 6)
                 for j in range(NJ)]

        def lo_of(t):
            return pl.multiple_of((wid + NW * t) * TCH, TCH)

        def read_start(t, h):
            pltpu.async_copy(tab_hbm.at[:, pl.ds(lo_of(t), TCH)],
                             tin.at[h], rsem.at[h])

        def read_wait(h):
            pltpu.make_async_copy(tab_hbm.at[:, pl.ds(0, TCH)],
                                  tin.at[h], rsem.at[h]).wait()

        def write_start(t):
            dst = pl.ds(pl.multiple_of(lo_of(t) // 2, TCH // 2), TCH // 2)
            pltpu.async_copy(tout, out_hbm.at[dst], wsem)

        def write_wait():
            pltpu.make_async_copy(tout, out_hbm.at[pl.ds(0, TCH // 2)],
                                  wsem).wait()

        def transpose(h):
            def dbody(d, carry):
                vals = [tin[h, d, pl.ds(16 * j, 16)] for j in range(NJ)]
                for j in range(NJ):
                    plsc.store_scatter(tout, [pvecs[j], ovecs[j] + d],
                                       vals[j])
                return carry
            lax.fori_loop(0, EMBED_DIM, dbody, 0)

        def do_chunk(t, h, first):
            read_wait(h)
            if not first:
                write_wait()
            transpose(h)
            write_start(t)

            @pl.when(t + 2 < nw_chunks)
            def _():
                read_start(t + 2, h)

        read_start(0, 0)
        read_start(1, 1)

        def body(p, carry):
            t0 = 2 * p

            @pl.when(t0 < nw_chunks)
            def _():
                do_chunk(t0, 0, False)

            @pl.when(t0 + 1 < nw_chunks)
            def _():
                do_chunk(t0 + 1, 1, False)

            return carry

        do_chunk(0, 0, True)
        do_chunk(1, 1, False)
        lax.fori_loop(1, (NFULL // NW + 2) // 2, body, 0)
        write_wait()

        @pl.when(wid == 0)
        def _():
            lo = NFULL * TCH
            pltpu.sync_copy(tail_hbm, ttail)

            def tbody(d, carry):
                for j in range(4):
                    v = ttail[d, pl.ds(16 * j, 16)]
                    plsc.store_scatter(tout, [pvecs[j], ovecs[j] + d], v)
                return carry
            lax.fori_loop(0, EMBED_DIM, tbody, 0)
            pltpu.sync_copy(tout.at[pl.ds(0, 32)],
                            out_hbm.at[pl.ds(lo // 2, 32)])

    packed = k1(tab_t, tail_t).reshape(VOCAB, EMBED_DIM)

    @functools.partial(
        pl.kernel,
        mesh=_sc_mesh(),
        out_type=jax.ShapeDtypeStruct((SEQ_LEN, BATCH, EMBED_DIM),
                                      jnp.float32),
        scratch_types=[
            pltpu.VMEM((NUNIT, BT), jnp.int32),
            pltpu.VMEM((GN, BT, EMBED_DIM), jnp.float32),
            pltpu.SemaphoreType.DMA((GN,)),
            pltpu.SemaphoreType.DMA((GN,)),
        ],
        compiler_params=pltpu.CompilerParams(
            use_tc_tiling_on_sc=False, needs_layout_passes=False),
    )
    def k2(text_hbm, tab_hbm, out_hbm, idx_v, grows, gsem, wsem):
        wid = lax.axis_index("s") * NUM_CORES + lax.axis_index("c")
        b0 = wid * BT

        pltpu.sync_copy(text_hbm.at[:, pl.ds(b0, BT)], idx_v)

        def gather_start(u, h):
            pltpu.async_copy(tab_hbm.at[idx_v.at[u]], grows.at[h],
                             gsem.at[h])

        def gather_wait(h):
            pltpu.make_async_copy(tab_hbm.at[idx_v.at[0]], grows.at[h],
                                  gsem.at[h]).wait()

        def write_start(u, h):
            pltpu.async_copy(grows.at[h],
                             out_hbm.at[u, pl.ds(b0, BT)],
                             wsem.at[h])

        def write_wait(h):
            pltpu.make_async_copy(grows.at[h],
                                  out_hbm.at[0, pl.ds(b0, BT)],
                                  wsem.at[h]).wait()

        for h in range(GN):
            gather_start(h, h)

        def body(p, carry):
            for h in range(GN):
                u = GN * p + h
                hp = (h - 1) % GN
                gather_wait(h)

                @pl.when(u >= 1)
                def _():
                    write_wait(hp)

                    @pl.when(u + GN - 1 < NUNIT)
                    def _():
                        gather_start(u + GN - 1, hp)

                write_start(u, h)

            return carry

        lax.fori_loop(0, NUNIT // GN, body, 0)
        write_wait((NUNIT - 1) % GN)

    return k2(text_t, packed)


def kernel(text, table):
    tail_t = jnp.pad(table[VOCAB - TAILW:, :].T, ((0, 0), (0, 128 - TAILW)))
    return _embed(text.T, table.T, tail_t).transpose(1, 0, 2)


# speedup vs baseline: 4.4764x; 1.1792x over previous
import functools

import jax
import jax.numpy as jnp
from jax import lax
from jax.experimental import pallas as pl
from jax.experimental.pallas import tpu as pltpu
from jax.experimental.pallas import tpu_sc as plsc

VOCAB = 1000000
EMBED_DIM = 64
BATCH = 4096
SEQ_LEN = 200

NUM_CORES = 2
NUM_SUBCORES = 16
NW = NUM_CORES * NUM_SUBCORES

TCH = 512
NFULL = VOCAB // TCH
TAILW = VOCAB - NFULL * TCH
BT = BATCH // NW
NUNIT = SEQ_LEN
GN = 4


def _sc_mesh():
    return plsc.VectorSubcoreMesh(
        core_axis_name="c", subcore_axis_name="s",
        num_cores=NUM_CORES, num_subcores=NUM_SUBCORES)


@jax.jit
def _embed(text_t, tab_t, tail_t):
    @functools.partial(
        pl.kernel,
        mesh=_sc_mesh(),
        out_type=jax.ShapeDtypeStruct((VOCAB // 2, 128), jnp.float32),
        scratch_types=[
            pltpu.VMEM((2, EMBED_DIM, TCH), jnp.float32),
            pltpu.VMEM((TCH // 2, 128), jnp.float32),
            pltpu.VMEM((EMBED_DIM, 128), jnp.float32),
            pltpu.SemaphoreType.DMA((2,)),
            pltpu.SemaphoreType.DMA,
        ],
        compiler_params=pltpu.CompilerParams(
            use_tc_tiling_on_sc=True, needs_layout_passes=False),
    )
    def k1(tab_hbm, tail_hbm, out_hbm, tin, tout, ttail, rsem, wsem):
        wid = lax.axis_index("s") * NUM_CORES + lax.axis_index("c")
        nw_chunks = (NFULL - wid + NW - 1) // NW
        iota = lax.iota(jnp.int32, 16)
        NJ = TCH // 16
        pvecs = [lax.shift_right_logical(iota + 16 * j, 1) for j in range(NJ)]
        ovecs = [lax.shift_left(lax.bitwise_and(iota + 16 * j, 1), 6)
                 for j in range(NJ)]

        def lo_of(t):
            return pl.multiple_of((wid + NW * t) * TCH, TCH)

        def read_start(t, h):
            pltpu.async_copy(tab_hbm.at[:, pl.ds(lo_of(t), TCH)],
                             tin.at[h], rsem.at[h])

        def read_wait(h):
            pltpu.make_async_copy(tab_hbm.at[:, pl.ds(0, TCH)],
                                  tin.at[h], rsem.at[h]).wait()

        def write_start(t):
            dst = pl.ds(pl.multiple_of(lo_of(t) // 2, TCH // 2), TCH // 2)
            pltpu.async_copy(tout, out_hbm.at[dst], wsem)

        def write_wait():
            pltpu.make_async_copy(tout, out_hbm.at[pl.ds(0, TCH // 2)],
                                  wsem).wait()

        def transpose(h):
            def jbody(j, carry):
                jb = j * 16
                colv = iota + jb
                pvec = lax.shift_right_logical(colv, 1)
                ovec = lax.shift_left(lax.bitwise_and(colv, 1), 6)
                for db in range(0, EMBED_DIM, 16):
                    vals = [tin[h, db + r, pl.ds(jb, 16)] for r in range(16)]
                    for r in range(16):
                        plsc.store_scatter(tout, [pvec, ovec + (db + r)],
                                           vals[r])
                return carry
            lax.fori_loop(0, NJ, jbody, 0)

        def do_chunk(t, h, first):
            read_wait(h)
            if not first:
                write_wait()
            transpose(h)
            write_start(t)

            @pl.when(t + 2 < nw_chunks)
            def _():
                read_start(t + 2, h)

        read_start(0, 0)
        read_start(1, 1)

        def body(p, carry):
            t0 = 2 * p

            @pl.when(t0 < nw_chunks)
            def _():
                do_chunk(t0, 0, False)

            @pl.when(t0 + 1 < nw_chunks)
            def _():
                do_chunk(t0 + 1, 1, False)

            return carry

        do_chunk(0, 0, True)
        do_chunk(1, 1, False)
        lax.fori_loop(1, (NFULL // NW + 2) // 2, body, 0)
        write_wait()

        @pl.when(wid == 0)
        def _():
            lo = NFULL * TCH
            pltpu.sync_copy(tail_hbm, ttail)

            def tbody(d, carry):
                for j in range(4):
                    v = ttail[d, pl.ds(16 * j, 16)]
                    plsc.store_scatter(tout, [pvecs[j], ovecs[j] + d], v)
                return carry
            lax.fori_loop(0, EMBED_DIM, tbody, 0)
            pltpu.sync_copy(tout.at[pl.ds(0, 32)],
                            out_hbm.at[pl.ds(lo // 2, 32)])

    packed = k1(tab_t, tail_t).reshape(VOCAB, EMBED_DIM)

    @functools.partial(
        pl.kernel,
        mesh=_sc_mesh(),
        out_type=jax.ShapeDtypeStruct((SEQ_LEN, BATCH, EMBED_DIM),
                                      jnp.float32),
        scratch_types=[
            pltpu.VMEM((NUNIT, BT), jnp.int32),
            pltpu.VMEM((GN, BT, EMBED_DIM), jnp.float32),
            pltpu.SemaphoreType.DMA((GN,)),
            pltpu.SemaphoreType.DMA((GN,)),
        ],
        compiler_params=pltpu.CompilerParams(
            use_tc_tiling_on_sc=False, needs_layout_passes=False),
    )
    def k2(text_hbm, tab_hbm, out_hbm, idx_v, grows, gsem, wsem):
        wid = lax.axis_index("s") * NUM_CORES + lax.axis_index("c")
        b0 = wid * BT

        pltpu.sync_copy(text_hbm.at[:, pl.ds(b0, BT)], idx_v)

        def gather_start(u, h):
            pltpu.async_copy(tab_hbm.at[idx_v.at[u]], grows.at[h],
                             gsem.at[h])

        def gather_wait(h):
            pltpu.make_async_copy(tab_hbm.at[idx_v.at[0]], grows.at[h],
                                  gsem.at[h]).wait()

        def write_start(u, h):
            pltpu.async_copy(grows.at[h],
                             out_hbm.at[u, pl.ds(b0, BT)],
                             wsem.at[h])

        def write_wait(h):
            pltpu.make_async_copy(grows.at[h],
                                  out_hbm.at[0, pl.ds(b0, BT)],
                                  wsem.at[h]).wait()

        for h in range(GN):
            gather_start(h, h)

        def body(p, carry):
            for h in range(GN):
                u = GN * p + h
                hp = (h - 1) % GN
                gather_wait(h)

                @pl.when(u >= 1)
                def _():
                    write_wait(hp)

                    @pl.when(u + GN - 1 < NUNIT)
                    def _():
                        gather_start(u + GN - 1, hp)

                write_start(u, h)

            return carry

        lax.fori_loop(0, NUNIT // GN, body, 0)
        write_wait((NUNIT - 1) % GN)

    return k2(text_t, packed)


def kernel(text, table):
    tail_t = jnp.pad(table[VOCAB - TAILW:, :].T, ((0, 0), (0, 128 - TAILW)))
    return _embed(text.T, table.T, tail_t).transpose(1, 0, 2)
